# baseline (speedup 1.0000x reference)
import jax
import jax.numpy as jnp
from jax.experimental import pallas as pl

K = 1024
D = 64
N = 16 * 576
ROWS_PER_BLOCK = 1152
GRID = N // ROWS_PER_BLOCK


def _vq_kernel(x_ref, w_ref, out_ref):
    x = x_ref[...]
    w = w_ref[...]
    zz = jnp.sum(x * x, axis=1, keepdims=True)
    ww = jnp.sum(w * w, axis=1)[None, :]
    dot = jax.lax.dot_general(
        x, w,
        dimension_numbers=(((1,), (1,)), ((), ())),
        preferred_element_type=jnp.float32,
    )
    dists = zz - 2.0 * dot + ww
    m = jnp.min(dists, axis=1, keepdims=True)
    iota = jax.lax.broadcasted_iota(jnp.int32, dists.shape, 1)
    idx = jnp.min(jnp.where(dists == m, iota, K), axis=1)
    out_ref[0, 0, :] = idx


def kernel(z_e_x, weight):
    B, T, Dd = z_e_x.shape
    flat = z_e_x.reshape(N, D)
    out = pl.pallas_call(
        _vq_kernel,
        grid=(GRID,),
        in_specs=[
            pl.BlockSpec((ROWS_PER_BLOCK, D), lambda i: (i, 0)),
            pl.BlockSpec((K, D), lambda i: (0, 0)),
        ],
        out_specs=pl.BlockSpec((1, 1, ROWS_PER_BLOCK), lambda i: (i, 0, 0)),
        out_shape=jax.ShapeDtypeStruct((GRID, 1, ROWS_PER_BLOCK), jnp.int32),
    )(flat, weight)
    return out.reshape(B, T)


# speedup vs baseline: 1.4659x; 1.4659x over previous
import jax
import jax.numpy as jnp
from jax.experimental import pallas as pl

K = 1024
D = 64
N = 16 * 576
ROWS_PER_BLOCK = 1152
GRID = N // ROWS_PER_BLOCK


CHUNK = 128


def _vq_kernel(x_ref, w_ref, out_ref):
    x = x_ref[...]
    R = x.shape[0]
    zz = jnp.sum(x * x, axis=1, keepdims=True)
    w = w_ref[...]
    ww = jax.lax.dot_general(
        jnp.ones((1, D), jnp.float32), w * w,
        dimension_numbers=(((1,), (1,)), ((), ())),
        preferred_element_type=jnp.float32,
    )
    bestv = bestk = None
    for k in range(K // CHUNK):
        wk = w[k * CHUNK:(k + 1) * CHUNK, :]
        wwk = ww[:, k * CHUNK:(k + 1) * CHUNK]
        dotk = jax.lax.dot_general(
            x, wk,
            dimension_numbers=(((1,), (1,)), ((), ())),
            preferred_element_type=jnp.float32,
        )
        vk = zz - 2.0 * dotk + wwk
        if bestv is None:
            bestv, bestk = vk, jnp.zeros((R, CHUNK), jnp.float32)
        else:
            lt = vk < bestv
            bestv = jnp.where(lt, vk, bestv)
            bestk = jnp.where(lt, float(k), bestk)
    lanef = jax.lax.broadcasted_iota(jnp.int32, (R, CHUNK), 1).astype(jnp.float32)
    cand = bestk * float(CHUNK) + lanef
    m = jnp.min(bestv, axis=1, keepdims=True)
    idxf = jnp.min(jnp.where(bestv == m, cand, float(K)), axis=1, keepdims=True)
    out_ref[...] = idxf.astype(jnp.int32)


def kernel(z_e_x, weight):
    B, T, Dd = z_e_x.shape
    flat = z_e_x.reshape(N, D)
    out = pl.pallas_call(
        _vq_kernel,
        grid=(GRID,),
        in_specs=[
            pl.BlockSpec((ROWS_PER_BLOCK, D), lambda i: (i, 0)),
            pl.BlockSpec((K, D), lambda i: (0, 0)),
        ],
        out_specs=pl.BlockSpec((ROWS_PER_BLOCK, 1), lambda i: (i, 0)),
        out_shape=jax.ShapeDtypeStruct((N, 1), jnp.int32),
    )(flat, weight)
    return out.reshape(B, T)


# speedup vs baseline: 1.4710x; 1.0035x over previous
import jax
import jax.numpy as jnp
from jax.experimental import pallas as pl

K = 1024
D = 64
N = 16 * 576
ROWS_PER_BLOCK = 2304
GRID = N // ROWS_PER_BLOCK


CHUNK = 128


def _vq_kernel(x_ref, w_ref, out_ref):
    x = x_ref[...]
    R = x.shape[0]
    zz = jnp.sum(x * x, axis=1, keepdims=True)
    w = w_ref[...]
    ww = jax.lax.dot_general(
        jnp.ones((1, D), jnp.float32), w * w,
        dimension_numbers=(((1,), (1,)), ((), ())),
        preferred_element_type=jnp.float32,
    )
    bestv = bestk = None
    for k in range(K // CHUNK):
        wk = w[k * CHUNK:(k + 1) * CHUNK, :]
        wwk = ww[:, k * CHUNK:(k + 1) * CHUNK]
        dotk = jax.lax.dot_general(
            x, wk,
            dimension_numbers=(((1,), (1,)), ((), ())),
            preferred_element_type=jnp.float32,
        )
        vk = zz - 2.0 * dotk + wwk
        if bestv is None:
            bestv, bestk = vk, jnp.zeros((R, CHUNK), jnp.float32)
        else:
            lt = vk < bestv
            bestv = jnp.where(lt, vk, bestv)
            bestk = jnp.where(lt, float(k), bestk)
    lanef = jax.lax.broadcasted_iota(jnp.int32, (R, CHUNK), 1).astype(jnp.float32)
    cand = bestk * float(CHUNK) + lanef
    m = jnp.min(bestv, axis=1, keepdims=True)
    idxf = jnp.min(jnp.where(bestv == m, cand, float(K)), axis=1, keepdims=True)
    out_ref[...] = idxf.astype(jnp.int32)


def kernel(z_e_x, weight):
    B, T, Dd = z_e_x.shape
    flat = z_e_x.reshape(N, D)
    out = pl.pallas_call(
        _vq_kernel,
        grid=(GRID,),
        in_specs=[
            pl.BlockSpec((ROWS_PER_BLOCK, D), lambda i: (i, 0)),
            pl.BlockSpec((K, D), lambda i: (0, 0)),
        ],
        out_specs=pl.BlockSpec((ROWS_PER_BLOCK, 1), lambda i: (i, 0)),
        out_shape=jax.ShapeDtypeStruct((N, 1), jnp.int32),
    )(flat, weight)
    return out.reshape(B, T)


# speedup vs baseline: 1.7372x; 1.1809x over previous
import jax
import jax.numpy as jnp
from jax.experimental import pallas as pl

K = 1024
D = 64
B = 16
T = 576
CHUNK = 128
BATCH_BLK = 8
R = BATCH_BLK * T


def _vq_kernel(x_ref, w_ref, out_ref):
    x = x_ref[...].reshape(R, D)
    zz = jnp.sum(x * x, axis=1, keepdims=True)
    w = w_ref[...]
    w2 = 2.0 * w
    ww = jax.lax.dot_general(
        jnp.ones((1, D), jnp.float32), w * w,
        dimension_numbers=(((1,), (1,)), ((), ())),
        preferred_element_type=jnp.float32,
    )
    bestv = bestk = None
    for k in range(K // CHUNK):
        w2k = w2[k * CHUNK:(k + 1) * CHUNK, :]
        wwk = ww[:, k * CHUNK:(k + 1) * CHUNK]
        dot2k = jax.lax.dot_general(
            x, w2k,
            dimension_numbers=(((1,), (1,)), ((), ())),
            preferred_element_type=jnp.float32,
        )
        vk = zz - dot2k + wwk
        if bestv is None:
            bestv, bestk = vk, jnp.zeros((R, CHUNK), jnp.float32)
        else:
            lt = vk < bestv
            bestv = jnp.where(lt, vk, bestv)
            bestk = jnp.where(lt, float(k), bestk)
    lanef = jax.lax.broadcasted_iota(jnp.int32, (R, CHUNK), 1).astype(jnp.float32)
    cand = bestk * float(CHUNK) + lanef
    m = jnp.min(bestv, axis=1, keepdims=True)
    idxf = jnp.min(jnp.where(bestv == m, cand, float(K)), axis=1, keepdims=True)
    idx = idxf.astype(jnp.int32)
    out_ref[...] = jax.lax.reshape(idx, (BATCH_BLK, T))


def kernel(z_e_x, weight):
    return pl.pallas_call(
        _vq_kernel,
        grid=(B // BATCH_BLK,),
        in_specs=[
            pl.BlockSpec((BATCH_BLK, T, D), lambda i: (i, 0, 0)),
            pl.BlockSpec((K, D), lambda i: (0, 0)),
        ],
        out_specs=pl.BlockSpec((BATCH_BLK, T), lambda i: (i, 0)),
        out_shape=jax.ShapeDtypeStruct((B, T), jnp.int32),
    )(z_e_x, weight)


# speedup vs baseline: 1.7691x; 1.0184x over previous
import jax
import jax.numpy as jnp
from jax.experimental import pallas as pl

K = 1024
D = 64
B = 16
T = 576
CHUNK = 128
BATCH_BLK = 8
R = BATCH_BLK * T


def _vq_kernel(x_ref, w_ref, out_ref):
    x = x_ref[...].reshape(R, D)
    zz = jnp.sum(x * x, axis=1, keepdims=True)
    w = w_ref[...]
    w2 = 2.0 * w
    ww = jax.lax.dot_general(
        jnp.ones((1, D), jnp.float32), w * w,
        dimension_numbers=(((1,), (1,)), ((), ())),
        preferred_element_type=jnp.float32,
    )
    vks = []
    for k in range(K // CHUNK):
        w2k = w2[k * CHUNK:(k + 1) * CHUNK, :]
        wwk = ww[:, k * CHUNK:(k + 1) * CHUNK]
        dot2k = jax.lax.dot_general(
            x, w2k,
            dimension_numbers=(((1,), (1,)), ((), ())),
            preferred_element_type=jnp.float32,
        )
        vks.append(zz - dot2k + wwk)
    level = vks
    while len(level) > 1:
        level = [jnp.minimum(level[i], level[i + 1]) for i in range(0, len(level), 2)]
    m = jnp.min(level[0], axis=1, keepdims=True)
    cur = jnp.full((R, CHUNK), float(K // CHUNK), jnp.float32)
    for k in reversed(range(K // CHUNK)):
        cur = jnp.where(vks[k] == m, float(k), cur)
    lanef = jax.lax.broadcasted_iota(jnp.int32, (R, CHUNK), 1).astype(jnp.float32)
    cand = jnp.where(cur < float(K // CHUNK),
                     cur * float(CHUNK) + lanef, float(K))
    idxf = jnp.min(cand, axis=1, keepdims=True)
    idx = idxf.astype(jnp.int32)
    out_ref[...] = jax.lax.reshape(idx, (BATCH_BLK, T))


def kernel(z_e_x, weight):
    return pl.pallas_call(
        _vq_kernel,
        grid=(B // BATCH_BLK,),
        in_specs=[
            pl.BlockSpec((BATCH_BLK, T, D), lambda i: (i, 0, 0)),
            pl.BlockSpec((K, D), lambda i: (0, 0)),
        ],
        out_specs=pl.BlockSpec((BATCH_BLK, T), lambda i: (i, 0)),
        out_shape=jax.ShapeDtypeStruct((B, T), jnp.int32),
    )(z_e_x, weight)
